# baseline (speedup 1.0000x reference)
import functools

import jax
import jax.numpy as jnp
from jax import lax
from jax.experimental import pallas as pl
from jax.experimental.pallas import tpu as pltpu, tpu_sc as plsc

D = 16
B = 16384
NC = 2
NS = 16
NW = NC * NS
B_PER_W = B // NW

_mesh = plsc.VectorSubcoreMesh(core_axis_name="c", subcore_axis_name="s")


@functools.partial(
    pl.kernel,
    mesh=_mesh,
    out_type=jax.ShapeDtypeStruct((B, D), jnp.float32),
    scratch_types=[
        pltpu.VMEM((B_PER_W,), jnp.int32),
        pltpu.VMEM((B_PER_W, D), jnp.float32),
        pltpu.SemaphoreType.DMA,
    ],
    compiler_params=pltpu.CompilerParams(use_tc_tiling_on_sc=False),
)
def _gather_kernel(idx_hbm, table_hbm, out_hbm, idx_v, rows_v, sem):
    wid = lax.axis_index("s") * NC + lax.axis_index("c")
    base = wid * B_PER_W
    pltpu.sync_copy(idx_hbm.at[pl.ds(base, B_PER_W)], idx_v)
    pltpu.async_copy(table_hbm.at[idx_v], rows_v, sem).wait()
    pltpu.sync_copy(rows_v, out_hbm.at[pl.ds(base, B_PER_W)])


def kernel(task_id, table):
    return _gather_kernel(task_id.astype(jnp.int32), table)


# speedup vs baseline: 1.0023x; 1.0023x over previous
import functools

import jax
import jax.numpy as jnp
from jax import lax
from jax.experimental import pallas as pl
from jax.experimental.pallas import tpu as pltpu, tpu_sc as plsc

D = 16
B = 16384
NC = 2
NS = 16
NW = NC * NS
B_PER_W = B // NW

_mesh = plsc.VectorSubcoreMesh(core_axis_name="c", subcore_axis_name="s")


@functools.partial(
    pl.kernel,
    mesh=_mesh,
    out_type=jax.ShapeDtypeStruct((B, D), jnp.float32),
    scratch_types=[
        pltpu.VMEM((B_PER_W,), jnp.int32),
        pltpu.VMEM((B_PER_W, D), jnp.float32),
        pltpu.SemaphoreType.DMA,
    ],
    compiler_params=pltpu.CompilerParams(
        use_tc_tiling_on_sc=False,
        disable_bounds_checks=True,
        disable_semaphore_checks=True,
        skip_device_barrier=True,
    ),
)
def _gather_kernel(idx_hbm, table_hbm, out_hbm, idx_v, rows_v, sem):
    wid = lax.axis_index("s") * NC + lax.axis_index("c")
    base = wid * B_PER_W
    pltpu.sync_copy(idx_hbm.at[pl.ds(base, B_PER_W)], idx_v)
    pltpu.async_copy(table_hbm.at[idx_v], rows_v, sem).wait()
    pltpu.sync_copy(rows_v, out_hbm.at[pl.ds(base, B_PER_W)])


def kernel(task_id, table):
    return _gather_kernel(task_id.astype(jnp.int32), table)


# speedup vs baseline: 1.9402x; 1.9358x over previous
import functools

import jax
import jax.numpy as jnp
from jax import lax
from jax.experimental import pallas as pl
from jax.experimental.pallas import tpu as pltpu, tpu_sc as plsc

D = 16
B = 16384
NC = 2
NS = 16
NW = NC * NS
B_PER_W = B // NW

_mesh = plsc.VectorSubcoreMesh(core_axis_name="c", subcore_axis_name="s")


@functools.partial(
    pl.kernel,
    mesh=_mesh,
    out_type=jax.ShapeDtypeStruct((D, B), jnp.float32),
    scratch_types=[
        pltpu.VMEM((B_PER_W,), jnp.int32),
        pltpu.VMEM((D, B_PER_W), jnp.float32),
        pltpu.SemaphoreType.DMA,
    ],
    compiler_params=pltpu.CompilerParams(
        use_tc_tiling_on_sc=False,
        disable_bounds_checks=True,
        disable_semaphore_checks=True,
        skip_device_barrier=True,
    ),
)
def _gather_kernel(idx_hbm, table_t_hbm, out_t_hbm, idx_v, buf_v, sem):
    wid = lax.axis_index("s") * NC + lax.axis_index("c")
    base = wid * B_PER_W
    pltpu.sync_copy(idx_hbm.at[pl.ds(base, B_PER_W)], idx_v)
    copies = [
        pltpu.async_copy(table_t_hbm.at[d].at[idx_v], buf_v.at[d], sem)
        for d in range(D)
    ]
    for c in copies:
        c.wait()
    pltpu.sync_copy(buf_v, out_t_hbm.at[:, pl.ds(base, B_PER_W)])


def kernel(task_id, table):
    out_t = _gather_kernel(task_id.astype(jnp.int32), table.T)
    return out_t.T


# speedup vs baseline: 2.0584x; 1.0609x over previous
import functools

import jax
import jax.numpy as jnp
from jax import lax
from jax.experimental import pallas as pl
from jax.experimental.pallas import tpu as pltpu, tpu_sc as plsc

D = 16
V = 100000
B = 16384
NC = 2
NS = 16
DH = D // NC
B_PER_T = B // NS
L = 16

_mesh = plsc.VectorSubcoreMesh(core_axis_name="c", subcore_axis_name="s")


@functools.partial(
    pl.kernel,
    mesh=_mesh,
    out_type=jax.ShapeDtypeStruct((D, B), jnp.float32),
    scratch_types=[
        pltpu.VMEM((B_PER_T,), jnp.int32),
        pltpu.VMEM((DH * B_PER_T,), jnp.int32),
        pltpu.VMEM((DH * B_PER_T,), jnp.float32),
        pltpu.VMEM_SHARED((DH * V,), jnp.float32),
        pltpu.SemaphoreType.DMA,
        pltpu.SemaphoreType.DMA,
    ],
    compiler_params=pltpu.CompilerParams(
        use_tc_tiling_on_sc=False,
        disable_bounds_checks=True,
        disable_semaphore_checks=True,
        skip_device_barrier=True,
    ),
)
def _gather_kernel(idx_hbm, table_t_hbm, out_t_hbm, idx_v, gidx_v, buf_v,
                   tbl_sp, ssem, gsem):
    cid = lax.axis_index("c")
    sid = lax.axis_index("s")
    row = cid * DH + sid // 2
    half = (sid % 2) * (V // 2)
    stage = pltpu.async_copy(
        table_t_hbm.at[row, pl.ds(half, V // 2)],
        tbl_sp.at[pl.ds((sid // 2) * V + half, V // 2)],
        ssem,
    )
    pltpu.sync_copy(idx_hbm.at[pl.ds(sid * B_PER_T, B_PER_T)], idx_v)

    def body(c, _):
        iv = idx_v[pl.ds(c * L, L)]
        for dd in range(DH):
            gidx_v[pl.ds(dd * B_PER_T + c * L, L)] = iv + dd * V
        return 0

    lax.fori_loop(0, B_PER_T // L, body, 0)
    stage.wait()
    plsc.subcore_barrier()
    copies = [
        pltpu.async_copy(
            tbl_sp.at[gidx_v.at[pl.ds(dd * B_PER_T + ch * 128, 128)]],
            buf_v.at[pl.ds(dd * B_PER_T + ch * 128, 128)],
            gsem,
        )
        for dd in range(DH)
        for ch in range(B_PER_T // 128)
    ]
    for c in copies:
        c.wait()
    for dd in range(DH):
        pltpu.sync_copy(
            buf_v.at[pl.ds(dd * B_PER_T, B_PER_T)],
            out_t_hbm.at[cid * DH + dd, pl.ds(sid * B_PER_T, B_PER_T)],
        )


def kernel(task_id, table):
    out_t = _gather_kernel(task_id.astype(jnp.int32), table.T)
    return out_t.T


# speedup vs baseline: 2.1064x; 1.0233x over previous
import functools

import jax
import jax.numpy as jnp
from jax import lax
from jax.experimental import pallas as pl
from jax.experimental.pallas import tpu as pltpu, tpu_sc as plsc

D = 16
V = 100000
B = 16384
NC = 2
NS = 16
DH = D // NC
B_PER_T = B // NS
L = 16

_mesh = plsc.VectorSubcoreMesh(core_axis_name="c", subcore_axis_name="s")


@functools.partial(
    pl.kernel,
    mesh=_mesh,
    out_type=jax.ShapeDtypeStruct((D, B), jnp.float32),
    scratch_types=[
        pltpu.VMEM((B_PER_T,), jnp.int32),
        pltpu.VMEM((DH * B_PER_T,), jnp.int32),
        pltpu.VMEM((DH * B_PER_T,), jnp.float32),
        pltpu.VMEM_SHARED((DH * V,), jnp.float32),
        pltpu.SemaphoreType.DMA,
        pltpu.SemaphoreType.DMA,
    ],
    compiler_params=pltpu.CompilerParams(
        use_tc_tiling_on_sc=False,
        disable_bounds_checks=True,
        disable_semaphore_checks=True,
        skip_device_barrier=True,
    ),
)
def _gather_kernel(idx_hbm, table_t_hbm, out_t_hbm, idx_v, gidx_v, buf_v,
                   tbl_sp, ssem, gsem):
    cid = lax.axis_index("c")
    sid = lax.axis_index("s")
    row = cid * DH + sid // 2
    half = (sid % 2) * (V // 2)
    stage = pltpu.async_copy(
        table_t_hbm.at[row, pl.ds(half, V // 2)],
        tbl_sp.at[pl.ds((sid // 2) * V + half, V // 2)],
        ssem,
    )
    pltpu.sync_copy(idx_hbm.at[pl.ds(sid * B_PER_T, B_PER_T)], idx_v)

    def body(c, _):
        iv = idx_v[pl.ds(c * L, L)]
        for dd in range(DH):
            gidx_v[pl.ds(dd * B_PER_T + c * L, L)] = iv + dd * V
        return 0

    lax.fori_loop(0, B_PER_T // L, body, 0)
    stage.wait()
    plsc.subcore_barrier()
    copies = [
        pltpu.async_copy(
            tbl_sp.at[gidx_v.at[pl.ds(dd * B_PER_T, B_PER_T)]],
            buf_v.at[pl.ds(dd * B_PER_T, B_PER_T)],
            gsem,
        )
        for dd in range(DH)
    ]
    outs = []
    for dd in range(DH):
        copies[dd].wait()
        outs.append(
            pltpu.async_copy(
                buf_v.at[pl.ds(dd * B_PER_T, B_PER_T)],
                out_t_hbm.at[cid * DH + dd, pl.ds(sid * B_PER_T, B_PER_T)],
                ssem,
            )
        )
    for o in outs:
        o.wait()


def kernel(task_id, table):
    out_t = _gather_kernel(task_id.astype(jnp.int32), table.T)
    return out_t.T


# speedup vs baseline: 2.2331x; 1.0601x over previous
import functools

import jax
import jax.numpy as jnp
from jax import lax
from jax.experimental import pallas as pl
from jax.experimental.pallas import tpu as pltpu, tpu_sc as plsc

D = 16
V = 100000
B = 16384
NC = 2
NS = 16
DH = D // NC
B_PER_T = B // NS
LB = B_PER_T // 128
L = 16

_mesh = plsc.VectorSubcoreMesh(core_axis_name="c", subcore_axis_name="s")


@functools.partial(
    pl.kernel,
    mesh=_mesh,
    out_type=jax.ShapeDtypeStruct((B * D // 128, 128), jnp.float32),
    scratch_types=[
        pltpu.VMEM((B_PER_T,), jnp.int32),
        pltpu.VMEM((DH * B_PER_T,), jnp.int32),
        pltpu.VMEM((LB, DH, 128), jnp.float32),
        pltpu.VMEM_SHARED((DH * V,), jnp.float32),
        pltpu.SemaphoreType.DMA,
        pltpu.SemaphoreType.DMA,
    ],
    compiler_params=pltpu.CompilerParams(
        use_tc_tiling_on_sc=False,
        disable_bounds_checks=True,
        disable_semaphore_checks=True,
        skip_device_barrier=True,
    ),
)
def _gather_kernel(idx_hbm, table_t_hbm, out_hbm, idx_v, gidx_v, buf_v,
                   tbl_sp, ssem, gsem):
    cid = lax.axis_index("c")
    sid = lax.axis_index("s")
    row = cid * DH + sid // 2
    half = (sid % 2) * (V // 2)
    stage = pltpu.async_copy(
        table_t_hbm.at[row, pl.ds(half, V // 2)],
        tbl_sp.at[pl.ds((sid // 2) * V + half, V // 2)],
        ssem,
    )
    pltpu.sync_copy(idx_hbm.at[pl.ds(sid * B_PER_T, B_PER_T)], idx_v)

    def body(c, _):
        iv = idx_v[pl.ds(c * L, L)]
        for dd in range(DH):
            gidx_v[pl.ds(dd * B_PER_T + c * L, L)] = iv + dd * V
        return 0

    lax.fori_loop(0, B_PER_T // L, body, 0)
    stage.wait()
    plsc.subcore_barrier()
    copies = [
        pltpu.async_copy(
            tbl_sp.at[gidx_v.at[pl.ds(dd * B_PER_T + lb * 128, 128)]],
            buf_v.at[lb, dd],
            gsem,
        )
        for dd in range(DH)
        for lb in range(LB)
    ]
    for c in copies:
        c.wait()
    outs = []
    for lb in range(LB):
        bb = cid * 128 + sid * LB + lb
        outs.append(
            pltpu.async_copy(
                buf_v.at[lb],
                out_hbm.at[pl.ds(bb * DH, DH), :],
                ssem,
            )
        )
    for o in outs:
        o.wait()


def kernel(task_id, table):
    out2048 = _gather_kernel(task_id.astype(jnp.int32), table.T)
    return (
        out2048.reshape(NC, B // 128, DH, 128)
        .transpose(1, 3, 0, 2)
        .reshape(B, D)
    )
